# baseline (speedup 1.0000x reference)
import functools

import jax
import jax.numpy as jnp
from jax import lax
from jax.experimental import pallas as pl
from jax.experimental.pallas import tpu as pltpu
from jax.experimental.pallas import tpu_sc as plsc

_NC = 2
_NS = 16
_L = 16
_NW = _NC * _NS
_CHUNK = 100
_ZCH = 128
_NBUF = 4


def _pad_n(n):
    m = _NS * _ZCH
    return ((n + m - 1) // m) * m


@functools.cache
def _make_seg_call(n, h, e, with_counts):
    npad = _pad_n(n)
    ew = e // _NW
    k = ew // _CHUNK
    rps = npad // _NS
    zc = rps // _ZCH
    tld = 5
    trows = n // tld

    mesh = plsc.VectorSubcoreMesh(
        core_axis_name="c", subcore_axis_name="s",
        num_cores=_NC, num_subcores=_NS)

    out_type = [jax.ShapeDtypeStruct((npad, _NC * h), jnp.float32)]
    scratch = [
        pltpu.VMEM((k, _CHUNK), jnp.int32),
        pltpu.VMEM((k, _CHUNK), jnp.int32),
        pltpu.VMEM((_NBUF, _CHUNK, h), jnp.float32),
        pltpu.VMEM_SHARED((npad, h), jnp.float32),
        [pltpu.SemaphoreType.DMA] * _NBUF,
        [pltpu.SemaphoreType.DMA] * _NBUF,
        pltpu.SemaphoreType.DMA,
    ]
    if with_counts:
        out_type.append(jax.ShapeDtypeStruct((_NC, npad, _L), jnp.float32))
        scratch += [
            pltpu.VMEM_SHARED((npad, _L), jnp.float32),
            pltpu.VMEM((_CHUNK, _L), jnp.float32),
        ]
    else:
        scratch += [
            pltpu.VMEM_SHARED((n, h), jnp.float32),
            pltpu.SemaphoreType.DMA,
        ]

    def body(p_hbm, src_hbm, dst_hbm, zrow_hbm, z16_hbm, ones_hbm, *rest):
        if with_counts:
            (out_hbm, cnt_hbm, src_v, dst_v, rows_v, acc,
             gsems, ssems, isem, cacc, ones_v) = rest
            tbl = p_hbm
        else:
            (out_hbm, src_v, dst_v, rows_v, acc,
             gsems, ssems, isem, tbl, tsem) = rest
        cid = lax.axis_index("c")
        sid = lax.axis_index("s")
        wid = cid * _NS + sid

        pltpu.async_copy(src_hbm.at[wid], src_v, isem)
        pltpu.async_copy(dst_hbm.at[wid], dst_v, isem)
        if with_counts:
            pltpu.sync_copy(ones_hbm, ones_v)
        if not with_counts:
            @pl.when(sid < tld)
            def _():
                tsl = pl.ds(sid * trows, trows)
                pltpu.async_copy(p_hbm.at[tsl], tbl.at[tsl], tsem)

        for t in range(zc):
            off = sid * rps + t * _ZCH
            pltpu.sync_copy(zrow_hbm, acc.at[pl.ds(off, _ZCH)])
            if with_counts:
                pltpu.sync_copy(z16_hbm, cacc.at[pl.ds(off, _ZCH)])
        if not with_counts:
            @pl.when(sid < tld)
            def _():
                tsl = pl.ds(sid * trows, trows)
                pltpu.make_async_copy(p_hbm.at[tsl], tbl.at[tsl],
                                      tsem).wait()
        plsc.subcore_barrier()

        pltpu.make_async_copy(src_hbm.at[wid], src_v, isem).wait()
        pltpu.make_async_copy(dst_hbm.at[wid], dst_v, isem).wait()
        pltpu.async_copy(tbl.at[src_v.at[0]], rows_v.at[0], gsems[0])
        pltpu.async_copy(tbl.at[src_v.at[1]], rows_v.at[1], gsems[1])

        def step(t, _):
            for b in range(_NBUF):
                j = _NBUF * t + b
                b2 = (b + 2) % _NBUF
                pltpu.make_async_copy(
                    tbl.at[src_v.at[j]], rows_v.at[b], gsems[b]).wait()
                pltpu.async_copy(
                    rows_v.at[b], acc.at[dst_v.at[j]], ssems[b], add=True)
                if with_counts:
                    pltpu.sync_copy(ones_v, cacc.at[dst_v.at[j]], add=True)

                @pl.when((j >= 2) & (j + 2 < k))
                def _():
                    pltpu.make_async_copy(
                        rows_v.at[b2], acc.at[dst_v.at[0]],
                        ssems[b2]).wait()

                @pl.when(j + 2 < k)
                def _():
                    pltpu.async_copy(
                        tbl.at[src_v.at[j + 2]], rows_v.at[b2], gsems[b2])
            return 0
        lax.fori_loop(0, k // _NBUF, step, 0)
        for b in range(_NBUF):
            pltpu.make_async_copy(
                rows_v.at[b], acc.at[dst_v.at[0]], ssems[b]).wait()
        plsc.subcore_barrier()

        sl = pl.ds(sid * rps, rps)
        for c in range(_NC):
            @pl.when(cid == c)
            def _():
                pltpu.sync_copy(acc.at[sl], out_hbm.at[sl, pl.ds(c * h, h)])
        if with_counts:
            pltpu.sync_copy(cacc.at[sl], cnt_hbm.at[cid, sl])

    return pl.kernel(body, out_type=out_type, mesh=mesh,
                     scratch_types=scratch,
                     compiler_params=pltpu.CompilerParams(
                         use_tc_tiling_on_sc=False))


_BN = 1000


def _dot_t(a, w):
    return lax.dot_general(a, w, (((1,), (1,)), ((), ())),
                           preferred_element_type=jnp.float32)


def _first_body(x_ref, wl_ref, wr_ref, bl_ref, p_ref, r_ref):
    x = x_ref[...]
    p_ref[...] = _dot_t(x, wl_ref[...])
    r_ref[...] = _dot_t(x, wr_ref[...]) + bl_ref[...]


def _mid_body(s_ref, ca_ref, cb_ref, r_ref, wl_ref, wr_ref, bl_ref,
              p_ref, rn_ref):
    cnt = ca_ref[0] + cb_ref[0]
    inv = 1.0 / jnp.maximum(cnt[:, 0:1], 1.0)
    sv = s_ref[...]
    hfeat = (sv[:, :64] + sv[:, 64:]) * inv + r_ref[...]
    hfeat = jnp.maximum(hfeat, 0.0)
    p_ref[...] = _dot_t(hfeat, wl_ref[...])
    rn_ref[...] = _dot_t(hfeat, wr_ref[...]) + bl_ref[...]


def _final_body(s_ref, ca_ref, cb_ref, r_ref, wlin_ref, blin_ref,
                z_ref):
    cnt = ca_ref[0] + cb_ref[0]
    inv = 1.0 / jnp.maximum(cnt[:, 0:1], 1.0)
    sv = s_ref[...]
    hfeat = (sv[:, :64] + sv[:, 64:]) * inv + r_ref[...]
    z_ref[...] = _dot_t(hfeat, wlin_ref[...]) + blin_ref[...]


def _row_spec(width):
    return pl.BlockSpec((_BN, width), lambda i: (i, 0))


def _part_spec(width, a):
    return pl.BlockSpec((1, _BN, width), lambda i, a=a: (a, i, 0))


def _full_spec(shape):
    return pl.BlockSpec(shape, lambda i: (0,) * len(shape))


@functools.cache
def _make_first_call(n, d, h):
    return pl.pallas_call(
        _first_body,
        grid=(n // _BN,),
        in_specs=[_row_spec(d), _full_spec((h, d)), _full_spec((h, d)),
                  _full_spec((1, h))],
        out_specs=[_row_spec(h), _row_spec(h)],
        out_shape=[jax.ShapeDtypeStruct((n, h), jnp.float32)] * 2,
    )


@functools.cache
def _make_mid_call(n, h):
    return pl.pallas_call(
        _mid_body,
        grid=(n // _BN,),
        in_specs=[_row_spec(2 * h), _part_spec(_L, 0),
                  _part_spec(_L, 1), _row_spec(h), _full_spec((h, h)),
                  _full_spec((h, h)), _full_spec((1, h))],
        out_specs=[_row_spec(h), _row_spec(h)],
        out_shape=[jax.ShapeDtypeStruct((n, h), jnp.float32)] * 2,
    )


@functools.cache
def _make_final_call(n, h, o):
    return pl.pallas_call(
        _final_body,
        grid=(n // _BN,),
        in_specs=[_row_spec(2 * h), _part_spec(_L, 0),
                  _part_spec(_L, 1), _row_spec(h), _full_spec((o, h)),
                  _full_spec((1, o))],
        out_specs=_row_spec(o),
        out_shape=jax.ShapeDtypeStruct((n, o), jnp.float32),
    )


@jax.jit
def kernel(x, edge_index, Wl1a, bl1a, Wr1a, Wl1b, bl1b, Wr1b,
           Wl2a, bl2a, Wr2a, Wl2b, bl2b, Wr2b, Wlin, blin):
    n, d = x.shape
    e = edge_index.shape[1]
    h = Wl1a.shape[0]
    o = Wlin.shape[0]

    ew = e // _NW
    src_r = edge_index[0].reshape(_NW, ew // _CHUNK, _CHUNK)
    dst_r = edge_index[1].reshape(_NW, ew // _CHUNK, _CHUNK)
    zrow = jnp.zeros((_ZCH, h), jnp.float32)
    z16 = jnp.zeros((_ZCH, _L), jnp.float32)
    ones = jnp.ones((_CHUNK, _L), jnp.float32)

    seg1 = _make_seg_call(n, h, e, True)
    seg = _make_seg_call(n, h, e, False)
    first = _make_first_call(n, d, h)
    mid = _make_mid_call(n, h)
    final = _make_final_call(n, h, o)

    p1, r1 = first(x, Wl1a, Wr1a, bl1a[None])
    s1, c1 = seg1(p1, src_r, dst_r, zrow, z16, ones)

    p2, r2 = mid(s1, c1, c1, r1, Wl1b, Wr1b, bl1b[None])
    (s2,) = seg(p2, src_r, dst_r, zrow, z16, ones)
    p3, r3 = mid(s2, c1, c1, r2, Wl2a, Wr2a, bl2a[None])
    (s3,) = seg(p3, src_r, dst_r, zrow, z16, ones)
    p4, r4 = mid(s3, c1, c1, r3, Wl2b, Wr2b, bl2b[None])
    (s4,) = seg(p4, src_r, dst_r, zrow, z16, ones)
    z = final(s4, c1, c1, r4, Wlin, blin[None])
    return z


# speedup vs baseline: 1.0120x; 1.0120x over previous
import functools

import jax
import jax.numpy as jnp
from jax import lax
from jax.experimental import pallas as pl
from jax.experimental.pallas import tpu as pltpu
from jax.experimental.pallas import tpu_sc as plsc

_NC = 2
_NS = 16
_L = 16
_NW = _NC * _NS
_CHUNK = 125
_ZCH = 128
_NBUF = 4


def _pad_n(n):
    m = _NS * _ZCH
    return ((n + m - 1) // m) * m


@functools.cache
def _make_seg_call(n, h, e, with_counts):
    npad = _pad_n(n)
    ew = e // _NW
    k = ew // _CHUNK
    rps = npad // _NS
    zc = rps // _ZCH
    tld = 5
    trows = n // tld

    mesh = plsc.VectorSubcoreMesh(
        core_axis_name="c", subcore_axis_name="s",
        num_cores=_NC, num_subcores=_NS)

    out_type = [jax.ShapeDtypeStruct((npad, _NC * h), jnp.float32)]
    scratch = [
        pltpu.VMEM((k, _CHUNK), jnp.int32),
        pltpu.VMEM((k, _CHUNK), jnp.int32),
        pltpu.VMEM((_NBUF, _CHUNK, h), jnp.float32),
        pltpu.VMEM_SHARED((npad, h), jnp.float32),
        [pltpu.SemaphoreType.DMA] * _NBUF,
        [pltpu.SemaphoreType.DMA] * _NBUF,
        pltpu.SemaphoreType.DMA,
    ]
    if with_counts:
        out_type.append(jax.ShapeDtypeStruct((_NC, npad, _L), jnp.float32))
        scratch += [
            pltpu.VMEM_SHARED((npad, _L), jnp.float32),
            pltpu.VMEM((_CHUNK, _L), jnp.float32),
        ]

    def body(p_hbm, src_hbm, dst_hbm, zrow_hbm, z16_hbm, ones_hbm, *rest):
        tbl = p_hbm
        if with_counts:
            (out_hbm, cnt_hbm, src_v, dst_v, rows_v, acc,
             gsems, ssems, isem, cacc, ones_v) = rest
        else:
            (out_hbm, src_v, dst_v, rows_v, acc,
             gsems, ssems, isem) = rest
        cid = lax.axis_index("c")
        sid = lax.axis_index("s")
        wid = cid * _NS + sid

        pltpu.async_copy(src_hbm.at[wid], src_v, isem)
        pltpu.async_copy(dst_hbm.at[wid], dst_v, isem)
        if with_counts:
            pltpu.sync_copy(ones_hbm, ones_v)

        for t in range(zc):
            off = sid * rps + t * _ZCH
            pltpu.sync_copy(zrow_hbm, acc.at[pl.ds(off, _ZCH)])
            if with_counts:
                pltpu.sync_copy(z16_hbm, cacc.at[pl.ds(off, _ZCH)])
        plsc.subcore_barrier()

        pltpu.make_async_copy(src_hbm.at[wid], src_v, isem).wait()
        pltpu.make_async_copy(dst_hbm.at[wid], dst_v, isem).wait()
        pltpu.async_copy(tbl.at[src_v.at[0]], rows_v.at[0], gsems[0])
        pltpu.async_copy(tbl.at[src_v.at[1]], rows_v.at[1], gsems[1])

        def step(t, _):
            for b in range(_NBUF):
                j = _NBUF * t + b
                b2 = (b + 2) % _NBUF
                pltpu.make_async_copy(
                    tbl.at[src_v.at[j]], rows_v.at[b], gsems[b]).wait()
                pltpu.async_copy(
                    rows_v.at[b], acc.at[dst_v.at[j]], ssems[b], add=True)
                if with_counts:
                    pltpu.sync_copy(ones_v, cacc.at[dst_v.at[j]], add=True)

                @pl.when((j >= 2) & (j + 2 < k))
                def _():
                    pltpu.make_async_copy(
                        rows_v.at[b2], acc.at[dst_v.at[0]],
                        ssems[b2]).wait()

                @pl.when(j + 2 < k)
                def _():
                    pltpu.async_copy(
                        tbl.at[src_v.at[j + 2]], rows_v.at[b2], gsems[b2])
            return 0
        lax.fori_loop(0, k // _NBUF, step, 0)
        for b in range(_NBUF):
            pltpu.make_async_copy(
                rows_v.at[b], acc.at[dst_v.at[0]], ssems[b]).wait()
        plsc.subcore_barrier()

        sl = pl.ds(sid * rps, rps)
        for c in range(_NC):
            @pl.when(cid == c)
            def _():
                pltpu.sync_copy(acc.at[sl], out_hbm.at[sl, pl.ds(c * h, h)])
        if with_counts:
            pltpu.sync_copy(cacc.at[sl], cnt_hbm.at[cid, sl])

    return pl.kernel(body, out_type=out_type, mesh=mesh,
                     scratch_types=scratch,
                     compiler_params=pltpu.CompilerParams(
                         use_tc_tiling_on_sc=False))


_BN = 1000


def _dot_t(a, w):
    return lax.dot_general(a, w, (((1,), (1,)), ((), ())),
                           preferred_element_type=jnp.float32)


def _first_body(x_ref, wl_ref, wr_ref, bl_ref, p_ref, r_ref):
    x = x_ref[...]
    p_ref[...] = _dot_t(x, wl_ref[...])
    r_ref[...] = _dot_t(x, wr_ref[...]) + bl_ref[...]


def _mid_body(s_ref, ca_ref, cb_ref, r_ref, wl_ref, wr_ref, bl_ref,
              p_ref, rn_ref):
    cnt = ca_ref[0] + cb_ref[0]
    inv = 1.0 / jnp.maximum(cnt[:, 0:1], 1.0)
    sv = s_ref[...]
    hfeat = (sv[:, :64] + sv[:, 64:]) * inv + r_ref[...]
    hfeat = jnp.maximum(hfeat, 0.0)
    p_ref[...] = _dot_t(hfeat, wl_ref[...])
    rn_ref[...] = _dot_t(hfeat, wr_ref[...]) + bl_ref[...]


def _final_body(s_ref, ca_ref, cb_ref, r_ref, wlin_ref, blin_ref,
                z_ref):
    cnt = ca_ref[0] + cb_ref[0]
    inv = 1.0 / jnp.maximum(cnt[:, 0:1], 1.0)
    sv = s_ref[...]
    hfeat = (sv[:, :64] + sv[:, 64:]) * inv + r_ref[...]
    z_ref[...] = _dot_t(hfeat, wlin_ref[...]) + blin_ref[...]


def _row_spec(width):
    return pl.BlockSpec((_BN, width), lambda i: (i, 0))


def _part_spec(width, a):
    return pl.BlockSpec((1, _BN, width), lambda i, a=a: (a, i, 0))


def _full_spec(shape):
    return pl.BlockSpec(shape, lambda i: (0,) * len(shape))


@functools.cache
def _make_first_call(n, d, h):
    return pl.pallas_call(
        _first_body,
        grid=(n // _BN,),
        in_specs=[_row_spec(d), _full_spec((h, d)), _full_spec((h, d)),
                  _full_spec((1, h))],
        out_specs=[_row_spec(h), _row_spec(h)],
        out_shape=[jax.ShapeDtypeStruct((n, h), jnp.float32)] * 2,
    )


@functools.cache
def _make_mid_call(n, h):
    return pl.pallas_call(
        _mid_body,
        grid=(n // _BN,),
        in_specs=[_row_spec(2 * h), _part_spec(_L, 0),
                  _part_spec(_L, 1), _row_spec(h), _full_spec((h, h)),
                  _full_spec((h, h)), _full_spec((1, h))],
        out_specs=[_row_spec(h), _row_spec(h)],
        out_shape=[jax.ShapeDtypeStruct((n, h), jnp.float32)] * 2,
    )


@functools.cache
def _make_final_call(n, h, o):
    return pl.pallas_call(
        _final_body,
        grid=(n // _BN,),
        in_specs=[_row_spec(2 * h), _part_spec(_L, 0),
                  _part_spec(_L, 1), _row_spec(h), _full_spec((o, h)),
                  _full_spec((1, o))],
        out_specs=_row_spec(o),
        out_shape=jax.ShapeDtypeStruct((n, o), jnp.float32),
    )


@jax.jit
def kernel(x, edge_index, Wl1a, bl1a, Wr1a, Wl1b, bl1b, Wr1b,
           Wl2a, bl2a, Wr2a, Wl2b, bl2b, Wr2b, Wlin, blin):
    n, d = x.shape
    e = edge_index.shape[1]
    h = Wl1a.shape[0]
    o = Wlin.shape[0]

    ew = e // _NW
    src_r = edge_index[0].reshape(_NW, ew // _CHUNK, _CHUNK)
    dst_r = edge_index[1].reshape(_NW, ew // _CHUNK, _CHUNK)
    zrow = jnp.zeros((_ZCH, h), jnp.float32)
    z16 = jnp.zeros((_ZCH, _L), jnp.float32)
    ones = jnp.ones((_CHUNK, _L), jnp.float32)

    seg1 = _make_seg_call(n, h, e, True)
    seg = _make_seg_call(n, h, e, False)
    first = _make_first_call(n, d, h)
    mid = _make_mid_call(n, h)
    final = _make_final_call(n, h, o)

    p1, r1 = first(x, Wl1a, Wr1a, bl1a[None])
    s1, c1 = seg1(p1, src_r, dst_r, zrow, z16, ones)

    p2, r2 = mid(s1, c1, c1, r1, Wl1b, Wr1b, bl1b[None])
    (s2,) = seg(p2, src_r, dst_r, zrow, z16, ones)
    p3, r3 = mid(s2, c1, c1, r2, Wl2a, Wr2a, bl2a[None])
    (s3,) = seg(p3, src_r, dst_r, zrow, z16, ones)
    p4, r4 = mid(s3, c1, c1, r3, Wl2b, Wr2b, bl2b[None])
    (s4,) = seg(p4, src_r, dst_r, zrow, z16, ones)
    z = final(s4, c1, c1, r4, Wlin, blin[None])
    return z


# speedup vs baseline: 1.0831x; 1.0702x over previous
import functools

import jax
import jax.numpy as jnp
from jax import lax
from jax.experimental import pallas as pl
from jax.experimental.pallas import tpu as pltpu
from jax.experimental.pallas import tpu_sc as plsc

_NC = 2
_NS = 16
_L = 16
_NW = _NC * _NS
_CHUNK = 125
_ZCH = 128
_NBUF = 4


def _pad_n(n):
    m = _NS * _ZCH
    return ((n + m - 1) // m) * m


@functools.cache
def _make_seg_call(n, h, e, with_counts):
    npad = _pad_n(n)
    ew = e // _NW
    k = ew // _CHUNK
    rps = npad // _NS
    zc = rps // _ZCH
    tld = 5
    trows = n // tld

    mesh = plsc.VectorSubcoreMesh(
        core_axis_name="c", subcore_axis_name="s",
        num_cores=_NC, num_subcores=_NS)

    out_type = [jax.ShapeDtypeStruct((npad, _NC * h), jnp.float32)]
    scratch = [
        pltpu.VMEM((k, _CHUNK), jnp.int32),
        pltpu.VMEM((k, _CHUNK), jnp.int32),
        pltpu.VMEM((_NBUF, _CHUNK, h), jnp.float32),
        pltpu.VMEM_SHARED((npad, h), jnp.float32),
        [pltpu.SemaphoreType.DMA] * _NBUF,
        [pltpu.SemaphoreType.DMA] * _NBUF,
        pltpu.SemaphoreType.DMA,
        pltpu.SemaphoreType.DMA,
    ]
    if with_counts:
        out_type.append(jax.ShapeDtypeStruct((_NC, npad, _L), jnp.float32))
        scratch += [
            pltpu.VMEM_SHARED((npad, _L), jnp.float32),
            pltpu.VMEM((_CHUNK, _L), jnp.float32),
        ]

    def body(p_hbm, src_hbm, dst_hbm, zrow_hbm, z16_hbm, ones_hbm, *rest):
        tbl = p_hbm
        if with_counts:
            (out_hbm, cnt_hbm, src_v, dst_v, rows_v, acc,
             gsems, ssems, isem, zsem, cacc, ones_v) = rest
        else:
            (out_hbm, src_v, dst_v, rows_v, acc,
             gsems, ssems, isem, zsem) = rest
        cid = lax.axis_index("c")
        sid = lax.axis_index("s")
        wid = cid * _NS + sid

        pltpu.async_copy(src_hbm.at[wid], src_v, isem)
        pltpu.async_copy(dst_hbm.at[wid], dst_v, isem)
        sl0 = pl.ds(sid * rps, rps)
        pltpu.async_copy(zrow_hbm, acc.at[sl0], zsem)
        if with_counts:
            pltpu.async_copy(z16_hbm, cacc.at[sl0], zsem)
            pltpu.async_copy(ones_hbm, ones_v, zsem)
        pltpu.make_async_copy(zrow_hbm, acc.at[sl0], zsem).wait()
        if with_counts:
            pltpu.make_async_copy(z16_hbm, cacc.at[sl0], zsem).wait()
            pltpu.make_async_copy(ones_hbm, ones_v, zsem).wait()
        plsc.subcore_barrier()

        pltpu.make_async_copy(src_hbm.at[wid], src_v, isem).wait()
        pltpu.make_async_copy(dst_hbm.at[wid], dst_v, isem).wait()
        pltpu.async_copy(tbl.at[src_v.at[0]], rows_v.at[0], gsems[0])
        pltpu.async_copy(tbl.at[src_v.at[1]], rows_v.at[1], gsems[1])

        def step(t, _):
            for b in range(_NBUF):
                j = _NBUF * t + b
                b2 = (b + 2) % _NBUF
                pltpu.make_async_copy(
                    tbl.at[src_v.at[j]], rows_v.at[b], gsems[b]).wait()
                pltpu.async_copy(
                    rows_v.at[b], acc.at[dst_v.at[j]], ssems[b], add=True)
                if with_counts:
                    pltpu.sync_copy(ones_v, cacc.at[dst_v.at[j]], add=True)

                @pl.when((j >= 2) & (j + 2 < k))
                def _():
                    pltpu.make_async_copy(
                        rows_v.at[b2], acc.at[dst_v.at[0]],
                        ssems[b2]).wait()

                @pl.when(j + 2 < k)
                def _():
                    pltpu.async_copy(
                        tbl.at[src_v.at[j + 2]], rows_v.at[b2], gsems[b2])
            return 0
        lax.fori_loop(0, k // _NBUF, step, 0)
        for b in range(_NBUF):
            pltpu.make_async_copy(
                rows_v.at[b], acc.at[dst_v.at[0]], ssems[b]).wait()
        plsc.subcore_barrier()

        sl = pl.ds(sid * rps, rps)
        for c in range(_NC):
            @pl.when(cid == c)
            def _():
                pltpu.sync_copy(acc.at[sl], out_hbm.at[sl, pl.ds(c * h, h)])
        if with_counts:
            pltpu.sync_copy(cacc.at[sl], cnt_hbm.at[cid, sl])

    return pl.kernel(body, out_type=out_type, mesh=mesh,
                     scratch_types=scratch,
                     compiler_params=pltpu.CompilerParams(
                         use_tc_tiling_on_sc=False))


_BN = 1000


def _dot_t(a, w):
    return lax.dot_general(a, w, (((1,), (1,)), ((), ())),
                           preferred_element_type=jnp.float32)


def _first_body(x_ref, wl_ref, wr_ref, bl_ref, p_ref, r_ref):
    x = x_ref[...]
    p_ref[...] = _dot_t(x, wl_ref[...])
    r_ref[...] = _dot_t(x, wr_ref[...]) + bl_ref[...]


def _mid_body(s_ref, ca_ref, cb_ref, r_ref, wl_ref, wr_ref, bl_ref,
              p_ref, rn_ref):
    cnt = ca_ref[0] + cb_ref[0]
    inv = 1.0 / jnp.maximum(cnt[:, 0:1], 1.0)
    sv = s_ref[...]
    hfeat = (sv[:, :64] + sv[:, 64:]) * inv + r_ref[...]
    hfeat = jnp.maximum(hfeat, 0.0)
    p_ref[...] = _dot_t(hfeat, wl_ref[...])
    rn_ref[...] = _dot_t(hfeat, wr_ref[...]) + bl_ref[...]


def _final_body(s_ref, ca_ref, cb_ref, r_ref, wlin_ref, blin_ref,
                z_ref):
    cnt = ca_ref[0] + cb_ref[0]
    inv = 1.0 / jnp.maximum(cnt[:, 0:1], 1.0)
    sv = s_ref[...]
    hfeat = (sv[:, :64] + sv[:, 64:]) * inv + r_ref[...]
    z_ref[...] = _dot_t(hfeat, wlin_ref[...]) + blin_ref[...]


def _row_spec(width):
    return pl.BlockSpec((_BN, width), lambda i: (i, 0))


def _part_spec(width, a):
    return pl.BlockSpec((1, _BN, width), lambda i, a=a: (a, i, 0))


def _full_spec(shape):
    return pl.BlockSpec(shape, lambda i: (0,) * len(shape))


@functools.cache
def _make_first_call(n, d, h):
    return pl.pallas_call(
        _first_body,
        grid=(n // _BN,),
        in_specs=[_row_spec(d), _full_spec((h, d)), _full_spec((h, d)),
                  _full_spec((1, h))],
        out_specs=[_row_spec(h), _row_spec(h)],
        out_shape=[jax.ShapeDtypeStruct((n, h), jnp.float32)] * 2,
    )


@functools.cache
def _make_mid_call(n, h):
    return pl.pallas_call(
        _mid_body,
        grid=(n // _BN,),
        in_specs=[_row_spec(2 * h), _part_spec(_L, 0),
                  _part_spec(_L, 1), _row_spec(h), _full_spec((h, h)),
                  _full_spec((h, h)), _full_spec((1, h))],
        out_specs=[_row_spec(h), _row_spec(h)],
        out_shape=[jax.ShapeDtypeStruct((n, h), jnp.float32)] * 2,
    )


@functools.cache
def _make_final_call(n, h, o):
    return pl.pallas_call(
        _final_body,
        grid=(n // _BN,),
        in_specs=[_row_spec(2 * h), _part_spec(_L, 0),
                  _part_spec(_L, 1), _row_spec(h), _full_spec((o, h)),
                  _full_spec((1, o))],
        out_specs=_row_spec(o),
        out_shape=jax.ShapeDtypeStruct((n, o), jnp.float32),
    )


@jax.jit
def kernel(x, edge_index, Wl1a, bl1a, Wr1a, Wl1b, bl1b, Wr1b,
           Wl2a, bl2a, Wr2a, Wl2b, bl2b, Wr2b, Wlin, blin):
    n, d = x.shape
    e = edge_index.shape[1]
    h = Wl1a.shape[0]
    o = Wlin.shape[0]

    ew = e // _NW
    src_r = edge_index[0].reshape(_NW, ew // _CHUNK, _CHUNK)
    dst_r = edge_index[1].reshape(_NW, ew // _CHUNK, _CHUNK)
    rps = _pad_n(n) // _NS
    zrow = jnp.zeros((rps, h), jnp.float32)
    z16 = jnp.zeros((rps, _L), jnp.float32)
    ones = jnp.ones((_CHUNK, _L), jnp.float32)

    seg1 = _make_seg_call(n, h, e, True)
    seg = _make_seg_call(n, h, e, False)
    first = _make_first_call(n, d, h)
    mid = _make_mid_call(n, h)
    final = _make_final_call(n, h, o)

    p1, r1 = first(x, Wl1a, Wr1a, bl1a[None])
    s1, c1 = seg1(p1, src_r, dst_r, zrow, z16, ones)

    p2, r2 = mid(s1, c1, c1, r1, Wl1b, Wr1b, bl1b[None])
    (s2,) = seg(p2, src_r, dst_r, zrow, z16, ones)
    p3, r3 = mid(s2, c1, c1, r2, Wl2a, Wr2a, bl2a[None])
    (s3,) = seg(p3, src_r, dst_r, zrow, z16, ones)
    p4, r4 = mid(s3, c1, c1, r3, Wl2b, Wr2b, bl2b[None])
    (s4,) = seg(p4, src_r, dst_r, zrow, z16, ones)
    z = final(s4, c1, c1, r4, Wlin, blin[None])
    return z


# speedup vs baseline: 1.1201x; 1.0341x over previous
import functools

import jax
import jax.numpy as jnp
from jax import lax
from jax.experimental import pallas as pl
from jax.experimental.pallas import tpu as pltpu
from jax.experimental.pallas import tpu_sc as plsc

_NC = 2
_NS = 16
_L = 16
_NW = _NC * _NS
_CHUNK = 125
_ZCH = 128
_NBUF = 4


def _pad_n(n):
    m = _NS * _ZCH
    return ((n + m - 1) // m) * m


@functools.cache
def _make_seg_call(n, h, e, with_counts):
    npad = _pad_n(n)
    ew = e // _NW
    k = ew // _CHUNK
    rps = npad // _NS
    zc = rps // _ZCH
    tld = 5
    trows = n // tld

    mesh = plsc.VectorSubcoreMesh(
        core_axis_name="c", subcore_axis_name="s",
        num_cores=_NC, num_subcores=_NS)

    out_type = [jax.ShapeDtypeStruct((npad, _NC * h), jnp.float32)]
    scratch = [
        pltpu.VMEM((k, _CHUNK), jnp.int32),
        pltpu.VMEM((k, _CHUNK), jnp.int32),
        pltpu.VMEM((_NBUF, _CHUNK, h), jnp.float32),
        pltpu.VMEM((_ZCH, h), jnp.float32),
        pltpu.VMEM_SHARED((npad, h), jnp.float32),
        [pltpu.SemaphoreType.DMA] * _NBUF,
        [pltpu.SemaphoreType.DMA] * _NBUF,
        pltpu.SemaphoreType.DMA,
        pltpu.SemaphoreType.DMA,
    ]
    if with_counts:
        out_type.append(jax.ShapeDtypeStruct((_NC, npad, _L), jnp.float32))
        scratch += [
            pltpu.VMEM_SHARED((npad, _L), jnp.float32),
            pltpu.VMEM((_CHUNK, _L), jnp.float32),
            pltpu.VMEM((_ZCH, _L), jnp.float32),
        ]

    def body(p_hbm, src_hbm, dst_hbm, zrow_hbm, z16_hbm, ones_hbm, *rest):
        tbl = p_hbm
        if with_counts:
            (out_hbm, cnt_hbm, src_v, dst_v, rows_v, zrow_v, acc,
             gsems, ssems, isem, zsem, cacc, ones_v, zcnt_v) = rest
        else:
            (out_hbm, src_v, dst_v, rows_v, zrow_v, acc,
             gsems, ssems, isem, zsem) = rest
        cid = lax.axis_index("c")
        sid = lax.axis_index("s")
        wid = cid * _NS + sid

        pltpu.async_copy(src_hbm.at[wid], src_v, isem)
        pltpu.async_copy(dst_hbm.at[wid], dst_v, isem)
        if with_counts:
            pltpu.async_copy(ones_hbm, ones_v, zsem)

        def fill_z(i, _):
            for q in range(h // _L):
                zrow_v[i, pl.ds(q * _L, _L)] = jnp.zeros((_L,), jnp.float32)
            if with_counts:
                zcnt_v[i, pl.ds(0, _L)] = jnp.zeros((_L,), jnp.float32)
            return 0
        lax.fori_loop(0, _ZCH, fill_z, 0)

        for t in range(zc):
            off = sid * rps + t * _ZCH
            pltpu.sync_copy(zrow_v, acc.at[pl.ds(off, _ZCH)])
            if with_counts:
                pltpu.sync_copy(zcnt_v, cacc.at[pl.ds(off, _ZCH)])
        if with_counts:
            pltpu.make_async_copy(ones_hbm, ones_v, zsem).wait()
        plsc.subcore_barrier()

        pltpu.make_async_copy(src_hbm.at[wid], src_v, isem).wait()
        pltpu.make_async_copy(dst_hbm.at[wid], dst_v, isem).wait()
        pltpu.async_copy(tbl.at[src_v.at[0]], rows_v.at[0], gsems[0])
        pltpu.async_copy(tbl.at[src_v.at[1]], rows_v.at[1], gsems[1])

        def step(t, _):
            for b in range(_NBUF):
                j = _NBUF * t + b
                b2 = (b + 2) % _NBUF
                pltpu.make_async_copy(
                    tbl.at[src_v.at[j]], rows_v.at[b], gsems[b]).wait()
                pltpu.async_copy(
                    rows_v.at[b], acc.at[dst_v.at[j]], ssems[b], add=True)
                if with_counts:
                    pltpu.sync_copy(ones_v, cacc.at[dst_v.at[j]], add=True)

                @pl.when((j >= 2) & (j + 2 < k))
                def _():
                    pltpu.make_async_copy(
                        rows_v.at[b2], acc.at[dst_v.at[0]],
                        ssems[b2]).wait()

                @pl.when(j + 2 < k)
                def _():
                    pltpu.async_copy(
                        tbl.at[src_v.at[j + 2]], rows_v.at[b2], gsems[b2])
            return 0
        lax.fori_loop(0, k // _NBUF, step, 0)
        for b in range(_NBUF):
            pltpu.make_async_copy(
                rows_v.at[b], acc.at[dst_v.at[0]], ssems[b]).wait()
        plsc.subcore_barrier()

        sl = pl.ds(sid * rps, rps)
        for c in range(_NC):
            @pl.when(cid == c)
            def _():
                pltpu.sync_copy(acc.at[sl], out_hbm.at[sl, pl.ds(c * h, h)])
        if with_counts:
            pltpu.sync_copy(cacc.at[sl], cnt_hbm.at[cid, sl])

    return pl.kernel(body, out_type=out_type, mesh=mesh,
                     scratch_types=scratch,
                     compiler_params=pltpu.CompilerParams(
                         use_tc_tiling_on_sc=False))


_BN = 1000


def _dot_t(a, w):
    return lax.dot_general(a, w, (((1,), (1,)), ((), ())),
                           preferred_element_type=jnp.float32)


def _first_body(x_ref, wl_ref, wr_ref, bl_ref, p_ref, r_ref):
    x = x_ref[...]
    p_ref[...] = _dot_t(x, wl_ref[...])
    r_ref[...] = _dot_t(x, wr_ref[...]) + bl_ref[...]


def _mid_body(s_ref, ca_ref, cb_ref, r_ref, wl_ref, wr_ref, bl_ref,
              p_ref, rn_ref):
    cnt = ca_ref[0] + cb_ref[0]
    inv = 1.0 / jnp.maximum(cnt[:, 0:1], 1.0)
    sv = s_ref[...]
    hfeat = (sv[:, :64] + sv[:, 64:]) * inv + r_ref[...]
    hfeat = jnp.maximum(hfeat, 0.0)
    p_ref[...] = _dot_t(hfeat, wl_ref[...])
    rn_ref[...] = _dot_t(hfeat, wr_ref[...]) + bl_ref[...]


def _final_body(s_ref, ca_ref, cb_ref, r_ref, wlin_ref, blin_ref,
                z_ref):
    cnt = ca_ref[0] + cb_ref[0]
    inv = 1.0 / jnp.maximum(cnt[:, 0:1], 1.0)
    sv = s_ref[...]
    hfeat = (sv[:, :64] + sv[:, 64:]) * inv + r_ref[...]
    z_ref[...] = _dot_t(hfeat, wlin_ref[...]) + blin_ref[...]


def _row_spec(width):
    return pl.BlockSpec((_BN, width), lambda i: (i, 0))


def _part_spec(width, a):
    return pl.BlockSpec((1, _BN, width), lambda i, a=a: (a, i, 0))


def _full_spec(shape):
    return pl.BlockSpec(shape, lambda i: (0,) * len(shape))


@functools.cache
def _make_first_call(n, d, h):
    return pl.pallas_call(
        _first_body,
        grid=(n // _BN,),
        in_specs=[_row_spec(d), _full_spec((h, d)), _full_spec((h, d)),
                  _full_spec((1, h))],
        out_specs=[_row_spec(h), _row_spec(h)],
        out_shape=[jax.ShapeDtypeStruct((n, h), jnp.float32)] * 2,
    )


@functools.cache
def _make_mid_call(n, h):
    return pl.pallas_call(
        _mid_body,
        grid=(n // _BN,),
        in_specs=[_row_spec(2 * h), _part_spec(_L, 0),
                  _part_spec(_L, 1), _row_spec(h), _full_spec((h, h)),
                  _full_spec((h, h)), _full_spec((1, h))],
        out_specs=[_row_spec(h), _row_spec(h)],
        out_shape=[jax.ShapeDtypeStruct((n, h), jnp.float32)] * 2,
    )


@functools.cache
def _make_final_call(n, h, o):
    return pl.pallas_call(
        _final_body,
        grid=(n // _BN,),
        in_specs=[_row_spec(2 * h), _part_spec(_L, 0),
                  _part_spec(_L, 1), _row_spec(h), _full_spec((o, h)),
                  _full_spec((1, o))],
        out_specs=_row_spec(o),
        out_shape=jax.ShapeDtypeStruct((n, o), jnp.float32),
    )


@jax.jit
def kernel(x, edge_index, Wl1a, bl1a, Wr1a, Wl1b, bl1b, Wr1b,
           Wl2a, bl2a, Wr2a, Wl2b, bl2b, Wr2b, Wlin, blin):
    n, d = x.shape
    e = edge_index.shape[1]
    h = Wl1a.shape[0]
    o = Wlin.shape[0]

    ew = e // _NW
    src_r = edge_index[0].reshape(_NW, ew // _CHUNK, _CHUNK)
    dst_r = edge_index[1].reshape(_NW, ew // _CHUNK, _CHUNK)
    rps = _pad_n(n) // _NS
    zrow = jnp.zeros((rps, h), jnp.float32)
    z16 = jnp.zeros((rps, _L), jnp.float32)
    ones = jnp.ones((_CHUNK, _L), jnp.float32)

    seg1 = _make_seg_call(n, h, e, True)
    seg = _make_seg_call(n, h, e, False)
    first = _make_first_call(n, d, h)
    mid = _make_mid_call(n, h)
    final = _make_final_call(n, h, o)

    p1, r1 = first(x, Wl1a, Wr1a, bl1a[None])
    s1, c1 = seg1(p1, src_r, dst_r, zrow, z16, ones)

    p2, r2 = mid(s1, c1, c1, r1, Wl1b, Wr1b, bl1b[None])
    (s2,) = seg(p2, src_r, dst_r, zrow, z16, ones)
    p3, r3 = mid(s2, c1, c1, r2, Wl2a, Wr2a, bl2a[None])
    (s3,) = seg(p3, src_r, dst_r, zrow, z16, ones)
    p4, r4 = mid(s3, c1, c1, r3, Wl2b, Wr2b, bl2b[None])
    (s4,) = seg(p4, src_r, dst_r, zrow, z16, ones)
    z = final(s4, c1, c1, r4, Wlin, blin[None])
    return z


# speedup vs baseline: 1.2115x; 1.0817x over previous
import functools

import jax
import jax.numpy as jnp
from jax import lax
from jax.experimental import pallas as pl
from jax.experimental.pallas import tpu as pltpu
from jax.experimental.pallas import tpu_sc as plsc

_NC = 2
_NS = 16
_L = 16
_NW = _NC * _NS
_CHUNK = 125
_ZCH = 128


def _pad_n(n):
    m = _NS * _ZCH
    return ((n + m - 1) // m) * m


@functools.cache
def _make_seg_call(n, h, e, with_counts):
    npad = _pad_n(n)
    ew = e // _NW
    k = ew // _CHUNK
    nbuf = 4 if with_counts else 5
    la = 2 if with_counts else 3
    rps = npad // _NS
    zc = rps // _ZCH
    tld = 5
    trows = n // tld

    mesh = plsc.VectorSubcoreMesh(
        core_axis_name="c", subcore_axis_name="s",
        num_cores=_NC, num_subcores=_NS)

    out_type = [jax.ShapeDtypeStruct((npad, _NC * h), jnp.float32)]
    scratch = [
        pltpu.VMEM((k, _CHUNK), jnp.int32),
        pltpu.VMEM((k, _CHUNK), jnp.int32),
        pltpu.VMEM((nbuf, _CHUNK, h), jnp.float32),
        pltpu.VMEM((_ZCH, h), jnp.float32),
        pltpu.VMEM_SHARED((npad, h), jnp.float32),
        [pltpu.SemaphoreType.DMA] * nbuf,
        [pltpu.SemaphoreType.DMA] * nbuf,
        pltpu.SemaphoreType.DMA,
        pltpu.SemaphoreType.DMA,
    ]
    if with_counts:
        out_type.append(jax.ShapeDtypeStruct((_NC, npad, _L), jnp.float32))
        scratch += [
            pltpu.VMEM_SHARED((npad, _L), jnp.float32),
            pltpu.VMEM((_CHUNK, _L), jnp.float32),
            pltpu.VMEM((_ZCH, _L), jnp.float32),
        ]

    def body(p_hbm, src_hbm, dst_hbm, zrow_hbm, z16_hbm, ones_hbm, *rest):
        tbl = p_hbm
        if with_counts:
            (out_hbm, cnt_hbm, src_v, dst_v, rows_v, zrow_v, acc,
             gsems, ssems, isem, zsem, cacc, ones_v, zcnt_v) = rest
        else:
            (out_hbm, src_v, dst_v, rows_v, zrow_v, acc,
             gsems, ssems, isem, zsem) = rest
        cid = lax.axis_index("c")
        sid = lax.axis_index("s")
        wid = cid * _NS + sid

        pltpu.async_copy(src_hbm.at[wid], src_v, isem)
        pltpu.async_copy(dst_hbm.at[wid], dst_v, isem)
        if with_counts:
            pltpu.async_copy(ones_hbm, ones_v, zsem)

        def fill_z(i, _):
            for q in range(h // _L):
                zrow_v[i, pl.ds(q * _L, _L)] = jnp.zeros((_L,), jnp.float32)
            if with_counts:
                zcnt_v[i, pl.ds(0, _L)] = jnp.zeros((_L,), jnp.float32)
            return 0
        lax.fori_loop(0, _ZCH, fill_z, 0)

        for t in range(zc):
            off = sid * rps + t * _ZCH
            pltpu.sync_copy(zrow_v, acc.at[pl.ds(off, _ZCH)])
            if with_counts:
                pltpu.sync_copy(zcnt_v, cacc.at[pl.ds(off, _ZCH)])
        if with_counts:
            pltpu.make_async_copy(ones_hbm, ones_v, zsem).wait()
        plsc.subcore_barrier()

        pltpu.make_async_copy(src_hbm.at[wid], src_v, isem).wait()
        pltpu.make_async_copy(dst_hbm.at[wid], dst_v, isem).wait()
        for b0 in range(la):
            pltpu.async_copy(tbl.at[src_v.at[b0]], rows_v.at[b0], gsems[b0])

        def step(t, _):
            for b in range(nbuf):
                j = nbuf * t + b
                b2 = (b + la) % nbuf
                pltpu.make_async_copy(
                    tbl.at[src_v.at[j]], rows_v.at[b], gsems[b]).wait()
                pltpu.async_copy(
                    rows_v.at[b], acc.at[dst_v.at[j]], ssems[b], add=True)
                if with_counts:
                    pltpu.sync_copy(ones_v, cacc.at[dst_v.at[j]], add=True)

                @pl.when((j >= nbuf - la) & (j + la < k))
                def _():
                    pltpu.make_async_copy(
                        rows_v.at[b2], acc.at[dst_v.at[0]],
                        ssems[b2]).wait()

                @pl.when(j + la < k)
                def _():
                    pltpu.async_copy(
                        tbl.at[src_v.at[j + la]], rows_v.at[b2], gsems[b2])
            return 0
        lax.fori_loop(0, k // nbuf, step, 0)
        for b in range(nbuf):
            pltpu.make_async_copy(
                rows_v.at[b], acc.at[dst_v.at[0]], ssems[b]).wait()
        plsc.subcore_barrier()

        sl = pl.ds(sid * rps, rps)
        for c in range(_NC):
            @pl.when(cid == c)
            def _():
                pltpu.sync_copy(acc.at[sl], out_hbm.at[sl, pl.ds(c * h, h)])
        if with_counts:
            pltpu.sync_copy(cacc.at[sl], cnt_hbm.at[cid, sl])

    return pl.kernel(body, out_type=out_type, mesh=mesh,
                     scratch_types=scratch,
                     compiler_params=pltpu.CompilerParams(
                         use_tc_tiling_on_sc=False))


_BN = 1000


def _dot_t(a, w):
    return lax.dot_general(a, w, (((1,), (1,)), ((), ())),
                           preferred_element_type=jnp.float32)


def _first_body(x_ref, wl_ref, wr_ref, bl_ref, p_ref, r_ref):
    x = x_ref[...]
    p_ref[...] = _dot_t(x, wl_ref[...])
    r_ref[...] = _dot_t(x, wr_ref[...]) + bl_ref[...]


def _mid_body(s_ref, ca_ref, cb_ref, r_ref, wl_ref, wr_ref, bl_ref,
              p_ref, rn_ref):
    cnt = ca_ref[0] + cb_ref[0]
    inv = 1.0 / jnp.maximum(cnt[:, 0:1], 1.0)
    sv = s_ref[...]
    hfeat = (sv[:, :64] + sv[:, 64:]) * inv + r_ref[...]
    hfeat = jnp.maximum(hfeat, 0.0)
    p_ref[...] = _dot_t(hfeat, wl_ref[...])
    rn_ref[...] = _dot_t(hfeat, wr_ref[...]) + bl_ref[...]


def _final_body(s_ref, ca_ref, cb_ref, r_ref, wlin_ref, blin_ref,
                z_ref):
    cnt = ca_ref[0] + cb_ref[0]
    inv = 1.0 / jnp.maximum(cnt[:, 0:1], 1.0)
    sv = s_ref[...]
    hfeat = (sv[:, :64] + sv[:, 64:]) * inv + r_ref[...]
    z_ref[...] = _dot_t(hfeat, wlin_ref[...]) + blin_ref[...]


def _row_spec(width):
    return pl.BlockSpec((_BN, width), lambda i: (i, 0))


def _part_spec(width, a):
    return pl.BlockSpec((1, _BN, width), lambda i, a=a: (a, i, 0))


def _full_spec(shape):
    return pl.BlockSpec(shape, lambda i: (0,) * len(shape))


@functools.cache
def _make_first_call(n, d, h):
    return pl.pallas_call(
        _first_body,
        grid=(n // _BN,),
        in_specs=[_row_spec(d), _full_spec((h, d)), _full_spec((h, d)),
                  _full_spec((1, h))],
        out_specs=[_row_spec(h), _row_spec(h)],
        out_shape=[jax.ShapeDtypeStruct((n, h), jnp.float32)] * 2,
    )


@functools.cache
def _make_mid_call(n, h):
    return pl.pallas_call(
        _mid_body,
        grid=(n // _BN,),
        in_specs=[_row_spec(2 * h), _part_spec(_L, 0),
                  _part_spec(_L, 1), _row_spec(h), _full_spec((h, h)),
                  _full_spec((h, h)), _full_spec((1, h))],
        out_specs=[_row_spec(h), _row_spec(h)],
        out_shape=[jax.ShapeDtypeStruct((n, h), jnp.float32)] * 2,
    )


@functools.cache
def _make_final_call(n, h, o):
    return pl.pallas_call(
        _final_body,
        grid=(n // _BN,),
        in_specs=[_row_spec(2 * h), _part_spec(_L, 0),
                  _part_spec(_L, 1), _row_spec(h), _full_spec((o, h)),
                  _full_spec((1, o))],
        out_specs=_row_spec(o),
        out_shape=jax.ShapeDtypeStruct((n, o), jnp.float32),
    )


@jax.jit
def kernel(x, edge_index, Wl1a, bl1a, Wr1a, Wl1b, bl1b, Wr1b,
           Wl2a, bl2a, Wr2a, Wl2b, bl2b, Wr2b, Wlin, blin):
    n, d = x.shape
    e = edge_index.shape[1]
    h = Wl1a.shape[0]
    o = Wlin.shape[0]

    ew = e // _NW
    src_r = edge_index[0].reshape(_NW, ew // _CHUNK, _CHUNK)
    dst_r = edge_index[1].reshape(_NW, ew // _CHUNK, _CHUNK)
    rps = _pad_n(n) // _NS
    zrow = jnp.zeros((rps, h), jnp.float32)
    z16 = jnp.zeros((rps, _L), jnp.float32)
    ones = jnp.ones((_CHUNK, _L), jnp.float32)

    seg1 = _make_seg_call(n, h, e, True)
    seg = _make_seg_call(n, h, e, False)
    first = _make_first_call(n, d, h)
    mid = _make_mid_call(n, h)
    final = _make_final_call(n, h, o)

    p1, r1 = first(x, Wl1a, Wr1a, bl1a[None])
    s1, c1 = seg1(p1, src_r, dst_r, zrow, z16, ones)

    p2, r2 = mid(s1, c1, c1, r1, Wl1b, Wr1b, bl1b[None])
    (s2,) = seg(p2, src_r, dst_r, zrow, z16, ones)
    p3, r3 = mid(s2, c1, c1, r2, Wl2a, Wr2a, bl2a[None])
    (s3,) = seg(p3, src_r, dst_r, zrow, z16, ones)
    p4, r4 = mid(s3, c1, c1, r3, Wl2b, Wr2b, bl2b[None])
    (s4,) = seg(p4, src_r, dst_r, zrow, z16, ones)
    z = final(s4, c1, c1, r4, Wlin, blin[None])
    return z


# speedup vs baseline: 1.2271x; 1.0128x over previous
import functools

import jax
import jax.numpy as jnp
from jax import lax
from jax.experimental import pallas as pl
from jax.experimental.pallas import tpu as pltpu
from jax.experimental.pallas import tpu_sc as plsc

_NC = 2
_NS = 16
_L = 16
_NW = _NC * _NS
_CHUNK = 125
_ZCH = 128


def _pad_n(n):
    m = _NS * _ZCH
    return ((n + m - 1) // m) * m


@functools.cache
def _make_seg_call(n, h, e, with_counts):
    npad = _pad_n(n)
    ew = e // _NW
    k = ew // _CHUNK
    nbuf = 5
    la = 3
    rps = npad // _NS
    zc = rps // _ZCH
    tld = 5
    trows = n // tld

    mesh = plsc.VectorSubcoreMesh(
        core_axis_name="c", subcore_axis_name="s",
        num_cores=_NC, num_subcores=_NS)

    out_type = [jax.ShapeDtypeStruct((npad, _NC * h), jnp.float32)]
    scratch = [
        pltpu.VMEM((k, _CHUNK), jnp.int32),
        pltpu.VMEM((k, _CHUNK), jnp.int32),
        pltpu.VMEM((nbuf, _CHUNK, h), jnp.float32),
        pltpu.VMEM((_ZCH, h), jnp.float32),
        pltpu.VMEM_SHARED((npad, h), jnp.float32),
        [pltpu.SemaphoreType.DMA] * nbuf,
        [pltpu.SemaphoreType.DMA] * nbuf,
        pltpu.SemaphoreType.DMA,
        pltpu.SemaphoreType.DMA,
    ]
    if with_counts:
        out_type.append(jax.ShapeDtypeStruct((_NC, npad, _L), jnp.float32))
        scratch += [
            pltpu.VMEM_SHARED((npad, _L), jnp.float32),
            pltpu.VMEM((_CHUNK, _L), jnp.float32),
            pltpu.VMEM((_ZCH, _L), jnp.float32),
        ]

    def body(p_hbm, src_hbm, dst_hbm, zrow_hbm, z16_hbm, ones_hbm, *rest):
        tbl = p_hbm
        if with_counts:
            (out_hbm, cnt_hbm, src_v, dst_v, rows_v, zrow_v, acc,
             gsems, ssems, isem, zsem, cacc, ones_v, zcnt_v) = rest
        else:
            (out_hbm, src_v, dst_v, rows_v, zrow_v, acc,
             gsems, ssems, isem, zsem) = rest
        cid = lax.axis_index("c")
        sid = lax.axis_index("s")
        wid = cid * _NS + sid

        pltpu.async_copy(src_hbm.at[wid], src_v, isem)
        pltpu.async_copy(dst_hbm.at[wid], dst_v, isem)
        if with_counts:
            pltpu.async_copy(ones_hbm, ones_v, zsem)

        def fill_z(i, _):
            for q in range(h // _L):
                zrow_v[i, pl.ds(q * _L, _L)] = jnp.zeros((_L,), jnp.float32)
            if with_counts:
                zcnt_v[i, pl.ds(0, _L)] = jnp.zeros((_L,), jnp.float32)
            return 0
        lax.fori_loop(0, _ZCH, fill_z, 0)

        for t in range(zc):
            off = sid * rps + t * _ZCH
            pltpu.sync_copy(zrow_v, acc.at[pl.ds(off, _ZCH)])
            if with_counts:
                pltpu.sync_copy(zcnt_v, cacc.at[pl.ds(off, _ZCH)])
        if with_counts:
            pltpu.make_async_copy(ones_hbm, ones_v, zsem).wait()
        plsc.subcore_barrier()

        pltpu.make_async_copy(src_hbm.at[wid], src_v, isem).wait()
        pltpu.make_async_copy(dst_hbm.at[wid], dst_v, isem).wait()
        for b0 in range(la):
            pltpu.async_copy(tbl.at[src_v.at[b0]], rows_v.at[b0], gsems[b0])

        def step(t, _):
            for b in range(nbuf):
                j = nbuf * t + b
                b2 = (b + la) % nbuf
                pltpu.make_async_copy(
                    tbl.at[src_v.at[j]], rows_v.at[b], gsems[b]).wait()
                pltpu.async_copy(
                    rows_v.at[b], acc.at[dst_v.at[j]], ssems[b], add=True)
                if with_counts:
                    pltpu.sync_copy(ones_v, cacc.at[dst_v.at[j]], add=True)

                @pl.when((j >= nbuf - la) & (j + la < k))
                def _():
                    pltpu.make_async_copy(
                        rows_v.at[b2], acc.at[dst_v.at[0]],
                        ssems[b2]).wait()

                @pl.when(j + la < k)
                def _():
                    pltpu.async_copy(
                        tbl.at[src_v.at[j + la]], rows_v.at[b2], gsems[b2])
            return 0
        lax.fori_loop(0, k // nbuf, step, 0)
        for b in range(nbuf):
            pltpu.make_async_copy(
                rows_v.at[b], acc.at[dst_v.at[0]], ssems[b]).wait()
        plsc.subcore_barrier()

        sl = pl.ds(sid * rps, rps)
        for c in range(_NC):
            @pl.when(cid == c)
            def _():
                pltpu.sync_copy(acc.at[sl], out_hbm.at[sl, pl.ds(c * h, h)])
        if with_counts:
            pltpu.sync_copy(cacc.at[sl], cnt_hbm.at[cid, sl])

    return pl.kernel(body, out_type=out_type, mesh=mesh,
                     scratch_types=scratch,
                     compiler_params=pltpu.CompilerParams(
                         use_tc_tiling_on_sc=False))


_BN = 1000


def _dot_t(a, w):
    return lax.dot_general(a, w, (((1,), (1,)), ((), ())),
                           preferred_element_type=jnp.float32)


def _first_body(x_ref, wl_ref, wr_ref, bl_ref, p_ref, r_ref):
    x = x_ref[...]
    p_ref[...] = _dot_t(x, wl_ref[...])
    r_ref[...] = _dot_t(x, wr_ref[...]) + bl_ref[...]


def _mid_body(s_ref, ca_ref, cb_ref, r_ref, wl_ref, wr_ref, bl_ref,
              p_ref, rn_ref):
    cnt = ca_ref[0] + cb_ref[0]
    inv = 1.0 / jnp.maximum(cnt[:, 0:1], 1.0)
    sv = s_ref[...]
    hfeat = (sv[:, :64] + sv[:, 64:]) * inv + r_ref[...]
    hfeat = jnp.maximum(hfeat, 0.0)
    p_ref[...] = _dot_t(hfeat, wl_ref[...])
    rn_ref[...] = _dot_t(hfeat, wr_ref[...]) + bl_ref[...]


def _final_body(s_ref, ca_ref, cb_ref, r_ref, wlin_ref, blin_ref,
                z_ref):
    cnt = ca_ref[0] + cb_ref[0]
    inv = 1.0 / jnp.maximum(cnt[:, 0:1], 1.0)
    sv = s_ref[...]
    hfeat = (sv[:, :64] + sv[:, 64:]) * inv + r_ref[...]
    z_ref[...] = _dot_t(hfeat, wlin_ref[...]) + blin_ref[...]


def _row_spec(width):
    return pl.BlockSpec((_BN, width), lambda i: (i, 0))


def _part_spec(width, a):
    return pl.BlockSpec((1, _BN, width), lambda i, a=a: (a, i, 0))


def _full_spec(shape):
    return pl.BlockSpec(shape, lambda i: (0,) * len(shape))


@functools.cache
def _make_first_call(n, d, h):
    return pl.pallas_call(
        _first_body,
        grid=(n // _BN,),
        in_specs=[_row_spec(d), _full_spec((h, d)), _full_spec((h, d)),
                  _full_spec((1, h))],
        out_specs=[_row_spec(h), _row_spec(h)],
        out_shape=[jax.ShapeDtypeStruct((n, h), jnp.float32)] * 2,
    )


@functools.cache
def _make_mid_call(n, h):
    return pl.pallas_call(
        _mid_body,
        grid=(n // _BN,),
        in_specs=[_row_spec(2 * h), _part_spec(_L, 0),
                  _part_spec(_L, 1), _row_spec(h), _full_spec((h, h)),
                  _full_spec((h, h)), _full_spec((1, h))],
        out_specs=[_row_spec(h), _row_spec(h)],
        out_shape=[jax.ShapeDtypeStruct((n, h), jnp.float32)] * 2,
    )


@functools.cache
def _make_final_call(n, h, o):
    return pl.pallas_call(
        _final_body,
        grid=(n // _BN,),
        in_specs=[_row_spec(2 * h), _part_spec(_L, 0),
                  _part_spec(_L, 1), _row_spec(h), _full_spec((o, h)),
                  _full_spec((1, o))],
        out_specs=_row_spec(o),
        out_shape=jax.ShapeDtypeStruct((n, o), jnp.float32),
    )


@jax.jit
def kernel(x, edge_index, Wl1a, bl1a, Wr1a, Wl1b, bl1b, Wr1b,
           Wl2a, bl2a, Wr2a, Wl2b, bl2b, Wr2b, Wlin, blin):
    n, d = x.shape
    e = edge_index.shape[1]
    h = Wl1a.shape[0]
    o = Wlin.shape[0]

    ew = e // _NW
    src_r = edge_index[0].reshape(_NW, ew // _CHUNK, _CHUNK)
    dst_r = edge_index[1].reshape(_NW, ew // _CHUNK, _CHUNK)
    rps = _pad_n(n) // _NS
    zrow = jnp.zeros((rps, h), jnp.float32)
    z16 = jnp.zeros((rps, _L), jnp.float32)
    ones = jnp.ones((_CHUNK, _L), jnp.float32)

    seg1 = _make_seg_call(n, h, e, True)
    seg = _make_seg_call(n, h, e, False)
    first = _make_first_call(n, d, h)
    mid = _make_mid_call(n, h)
    final = _make_final_call(n, h, o)

    p1, r1 = first(x, Wl1a, Wr1a, bl1a[None])
    s1, c1 = seg1(p1, src_r, dst_r, zrow, z16, ones)

    p2, r2 = mid(s1, c1, c1, r1, Wl1b, Wr1b, bl1b[None])
    (s2,) = seg(p2, src_r, dst_r, zrow, z16, ones)
    p3, r3 = mid(s2, c1, c1, r2, Wl2a, Wr2a, bl2a[None])
    (s3,) = seg(p3, src_r, dst_r, zrow, z16, ones)
    p4, r4 = mid(s3, c1, c1, r3, Wl2b, Wr2b, bl2b[None])
    (s4,) = seg(p4, src_r, dst_r, zrow, z16, ones)
    z = final(s4, c1, c1, r4, Wlin, blin[None])
    return z


# speedup vs baseline: 1.2449x; 1.0145x over previous
import functools

import jax
import jax.numpy as jnp
from jax import lax
from jax.experimental import pallas as pl
from jax.experimental.pallas import tpu as pltpu
from jax.experimental.pallas import tpu_sc as plsc

_NC = 2
_NS = 16
_L = 16
_NW = _NC * _NS
_CHUNK = 125
_ZCH = 128


def _pad_n(n):
    m = _NS * _ZCH
    return ((n + m - 1) // m) * m


@functools.cache
def _make_seg_call(n, h, e, with_counts):
    npad = _pad_n(n)
    ew = e // _NW
    k = ew // _CHUNK
    nbuf = 5
    la = 4
    rps = npad // _NS
    zc = rps // _ZCH
    tld = 5
    trows = n // tld

    mesh = plsc.VectorSubcoreMesh(
        core_axis_name="c", subcore_axis_name="s",
        num_cores=_NC, num_subcores=_NS)

    out_type = [jax.ShapeDtypeStruct((npad, _NC * h), jnp.float32)]
    scratch = [
        pltpu.VMEM((k, _CHUNK), jnp.int32),
        pltpu.VMEM((k, _CHUNK), jnp.int32),
        pltpu.VMEM((nbuf, _CHUNK, h), jnp.float32),
        pltpu.VMEM((_ZCH, h), jnp.float32),
        pltpu.VMEM_SHARED((npad, h), jnp.float32),
        [pltpu.SemaphoreType.DMA] * nbuf,
        [pltpu.SemaphoreType.DMA] * nbuf,
        pltpu.SemaphoreType.DMA,
        pltpu.SemaphoreType.DMA,
    ]
    if with_counts:
        out_type.append(jax.ShapeDtypeStruct((_NC, npad, _L), jnp.float32))
        scratch += [
            pltpu.VMEM_SHARED((npad, _L), jnp.float32),
            pltpu.VMEM((_CHUNK, _L), jnp.float32),
            pltpu.VMEM((_ZCH, _L), jnp.float32),
        ]

    def body(p_hbm, src_hbm, dst_hbm, zrow_hbm, z16_hbm, ones_hbm, *rest):
        tbl = p_hbm
        if with_counts:
            (out_hbm, cnt_hbm, src_v, dst_v, rows_v, zrow_v, acc,
             gsems, ssems, isem, zsem, cacc, ones_v, zcnt_v) = rest
        else:
            (out_hbm, src_v, dst_v, rows_v, zrow_v, acc,
             gsems, ssems, isem, zsem) = rest
        cid = lax.axis_index("c")
        sid = lax.axis_index("s")
        wid = cid * _NS + sid

        pltpu.async_copy(src_hbm.at[wid], src_v, isem)
        pltpu.async_copy(dst_hbm.at[wid], dst_v, isem)
        if with_counts:
            pltpu.async_copy(ones_hbm, ones_v, zsem)

        def fill_z(i, _):
            for q in range(h // _L):
                zrow_v[i, pl.ds(q * _L, _L)] = jnp.zeros((_L,), jnp.float32)
            if with_counts:
                zcnt_v[i, pl.ds(0, _L)] = jnp.zeros((_L,), jnp.float32)
            return 0
        lax.fori_loop(0, _ZCH, fill_z, 0)

        for t in range(zc):
            off = sid * rps + t * _ZCH
            pltpu.sync_copy(zrow_v, acc.at[pl.ds(off, _ZCH)])
            if with_counts:
                pltpu.sync_copy(zcnt_v, cacc.at[pl.ds(off, _ZCH)])
        if with_counts:
            pltpu.make_async_copy(ones_hbm, ones_v, zsem).wait()
        plsc.subcore_barrier()

        pltpu.make_async_copy(src_hbm.at[wid], src_v, isem).wait()
        pltpu.make_async_copy(dst_hbm.at[wid], dst_v, isem).wait()
        for b0 in range(la):
            pltpu.async_copy(tbl.at[src_v.at[b0]], rows_v.at[b0], gsems[b0])

        def step(t, _):
            for b in range(nbuf):
                j = nbuf * t + b
                b2 = (b + la) % nbuf
                pltpu.make_async_copy(
                    tbl.at[src_v.at[j]], rows_v.at[b], gsems[b]).wait()
                pltpu.async_copy(
                    rows_v.at[b], acc.at[dst_v.at[j]], ssems[b], add=True)
                if with_counts:
                    pltpu.sync_copy(ones_v, cacc.at[dst_v.at[j]], add=True)

                @pl.when((j >= nbuf - la) & (j + la < k))
                def _():
                    pltpu.make_async_copy(
                        rows_v.at[b2], acc.at[dst_v.at[0]],
                        ssems[b2]).wait()

                @pl.when(j + la < k)
                def _():
                    pltpu.async_copy(
                        tbl.at[src_v.at[j + la]], rows_v.at[b2], gsems[b2])
            return 0
        lax.fori_loop(0, k // nbuf, step, 0)
        for b in range(nbuf):
            pltpu.make_async_copy(
                rows_v.at[b], acc.at[dst_v.at[0]], ssems[b]).wait()
        plsc.subcore_barrier()

        sl = pl.ds(sid * rps, rps)
        for c in range(_NC):
            @pl.when(cid == c)
            def _():
                pltpu.sync_copy(acc.at[sl], out_hbm.at[sl, pl.ds(c * h, h)])
        if with_counts:
            pltpu.sync_copy(cacc.at[sl], cnt_hbm.at[cid, sl])

    return pl.kernel(body, out_type=out_type, mesh=mesh,
                     scratch_types=scratch,
                     compiler_params=pltpu.CompilerParams(
                         use_tc_tiling_on_sc=False))


_BN = 1000


def _dot_t(a, w):
    return lax.dot_general(a, w, (((1,), (1,)), ((), ())),
                           preferred_element_type=jnp.float32)


def _first_body(x_ref, wl_ref, wr_ref, bl_ref, p_ref, r_ref):
    x = x_ref[...]
    p_ref[...] = _dot_t(x, wl_ref[...])
    r_ref[...] = _dot_t(x, wr_ref[...]) + bl_ref[...]


def _mid_body(s_ref, ca_ref, cb_ref, r_ref, wl_ref, wr_ref, bl_ref,
              p_ref, rn_ref):
    cnt = ca_ref[0] + cb_ref[0]
    inv = 1.0 / jnp.maximum(cnt[:, 0:1], 1.0)
    sv = s_ref[...]
    hfeat = (sv[:, :64] + sv[:, 64:]) * inv + r_ref[...]
    hfeat = jnp.maximum(hfeat, 0.0)
    p_ref[...] = _dot_t(hfeat, wl_ref[...])
    rn_ref[...] = _dot_t(hfeat, wr_ref[...]) + bl_ref[...]


def _final_body(s_ref, ca_ref, cb_ref, r_ref, wlin_ref, blin_ref,
                z_ref):
    cnt = ca_ref[0] + cb_ref[0]
    inv = 1.0 / jnp.maximum(cnt[:, 0:1], 1.0)
    sv = s_ref[...]
    hfeat = (sv[:, :64] + sv[:, 64:]) * inv + r_ref[...]
    z_ref[...] = _dot_t(hfeat, wlin_ref[...]) + blin_ref[...]


def _row_spec(width):
    return pl.BlockSpec((_BN, width), lambda i: (i, 0))


def _part_spec(width, a):
    return pl.BlockSpec((1, _BN, width), lambda i, a=a: (a, i, 0))


def _full_spec(shape):
    return pl.BlockSpec(shape, lambda i: (0,) * len(shape))


@functools.cache
def _make_first_call(n, d, h):
    return pl.pallas_call(
        _first_body,
        grid=(n // _BN,),
        in_specs=[_row_spec(d), _full_spec((h, d)), _full_spec((h, d)),
                  _full_spec((1, h))],
        out_specs=[_row_spec(h), _row_spec(h)],
        out_shape=[jax.ShapeDtypeStruct((n, h), jnp.float32)] * 2,
    )


@functools.cache
def _make_mid_call(n, h):
    return pl.pallas_call(
        _mid_body,
        grid=(n // _BN,),
        in_specs=[_row_spec(2 * h), _part_spec(_L, 0),
                  _part_spec(_L, 1), _row_spec(h), _full_spec((h, h)),
                  _full_spec((h, h)), _full_spec((1, h))],
        out_specs=[_row_spec(h), _row_spec(h)],
        out_shape=[jax.ShapeDtypeStruct((n, h), jnp.float32)] * 2,
    )


@functools.cache
def _make_final_call(n, h, o):
    return pl.pallas_call(
        _final_body,
        grid=(n // _BN,),
        in_specs=[_row_spec(2 * h), _part_spec(_L, 0),
                  _part_spec(_L, 1), _row_spec(h), _full_spec((o, h)),
                  _full_spec((1, o))],
        out_specs=_row_spec(o),
        out_shape=jax.ShapeDtypeStruct((n, o), jnp.float32),
    )


@jax.jit
def kernel(x, edge_index, Wl1a, bl1a, Wr1a, Wl1b, bl1b, Wr1b,
           Wl2a, bl2a, Wr2a, Wl2b, bl2b, Wr2b, Wlin, blin):
    n, d = x.shape
    e = edge_index.shape[1]
    h = Wl1a.shape[0]
    o = Wlin.shape[0]

    ew = e // _NW
    src_r = edge_index[0].reshape(_NW, ew // _CHUNK, _CHUNK)
    dst_r = edge_index[1].reshape(_NW, ew // _CHUNK, _CHUNK)
    rps = _pad_n(n) // _NS
    zrow = jnp.zeros((rps, h), jnp.float32)
    z16 = jnp.zeros((rps, _L), jnp.float32)
    ones = jnp.ones((_CHUNK, _L), jnp.float32)

    seg1 = _make_seg_call(n, h, e, True)
    seg = _make_seg_call(n, h, e, False)
    first = _make_first_call(n, d, h)
    mid = _make_mid_call(n, h)
    final = _make_final_call(n, h, o)

    p1, r1 = first(x, Wl1a, Wr1a, bl1a[None])
    s1, c1 = seg1(p1, src_r, dst_r, zrow, z16, ones)

    p2, r2 = mid(s1, c1, c1, r1, Wl1b, Wr1b, bl1b[None])
    (s2,) = seg(p2, src_r, dst_r, zrow, z16, ones)
    p3, r3 = mid(s2, c1, c1, r2, Wl2a, Wr2a, bl2a[None])
    (s3,) = seg(p3, src_r, dst_r, zrow, z16, ones)
    p4, r4 = mid(s3, c1, c1, r3, Wl2b, Wr2b, bl2b[None])
    (s4,) = seg(p4, src_r, dst_r, zrow, z16, ones)
    z = final(s4, c1, c1, r4, Wlin, blin[None])
    return z


# speedup vs baseline: 1.2467x; 1.0015x over previous
import functools

import jax
import jax.numpy as jnp
from jax import lax
from jax.experimental import pallas as pl
from jax.experimental.pallas import tpu as pltpu
from jax.experimental.pallas import tpu_sc as plsc

_NC = 2
_NS = 16
_L = 16
_NW = _NC * _NS
_CHUNK = 100
_ZCH = 128


def _pad_n(n):
    m = _NS * _ZCH
    return ((n + m - 1) // m) * m


@functools.cache
def _make_seg_call(n, h, e, with_counts):
    npad = _pad_n(n)
    ew = e // _NW
    k = ew // _CHUNK
    nbuf = 5
    la = 4
    rps = npad // _NS
    zc = rps // _ZCH
    tld = 5
    trows = n // tld

    mesh = plsc.VectorSubcoreMesh(
        core_axis_name="c", subcore_axis_name="s",
        num_cores=_NC, num_subcores=_NS)

    out_type = [jax.ShapeDtypeStruct((npad, _NC * h), jnp.float32)]
    scratch = [
        pltpu.VMEM((k, _CHUNK), jnp.int32),
        pltpu.VMEM((k, _CHUNK), jnp.int32),
        pltpu.VMEM((nbuf, _CHUNK, h), jnp.float32),
        pltpu.VMEM((_ZCH, h), jnp.float32),
        pltpu.VMEM_SHARED((npad, h), jnp.float32),
        [pltpu.SemaphoreType.DMA] * nbuf,
        [pltpu.SemaphoreType.DMA] * nbuf,
        pltpu.SemaphoreType.DMA,
        pltpu.SemaphoreType.DMA,
    ]
    if with_counts:
        out_type.append(jax.ShapeDtypeStruct((_NC, npad, _L), jnp.float32))
        scratch += [
            pltpu.VMEM_SHARED((npad, _L), jnp.float32),
            pltpu.VMEM((_CHUNK, _L), jnp.float32),
            pltpu.VMEM((_ZCH, _L), jnp.float32),
        ]

    def body(p_hbm, src_hbm, dst_hbm, zrow_hbm, z16_hbm, ones_hbm, *rest):
        tbl = p_hbm
        if with_counts:
            (out_hbm, cnt_hbm, src_v, dst_v, rows_v, zrow_v, acc,
             gsems, ssems, isem, zsem, cacc, ones_v, zcnt_v) = rest
        else:
            (out_hbm, src_v, dst_v, rows_v, zrow_v, acc,
             gsems, ssems, isem, zsem) = rest
        cid = lax.axis_index("c")
        sid = lax.axis_index("s")
        wid = cid * _NS + sid

        pltpu.async_copy(src_hbm.at[wid], src_v, isem)
        pltpu.async_copy(dst_hbm.at[wid], dst_v, isem)
        if with_counts:
            pltpu.async_copy(ones_hbm, ones_v, zsem)

        def fill_z(i, _):
            for q in range(h // _L):
                zrow_v[i, pl.ds(q * _L, _L)] = jnp.zeros((_L,), jnp.float32)
            if with_counts:
                zcnt_v[i, pl.ds(0, _L)] = jnp.zeros((_L,), jnp.float32)
            return 0
        lax.fori_loop(0, _ZCH, fill_z, 0)

        for t in range(zc):
            off = sid * rps + t * _ZCH
            pltpu.sync_copy(zrow_v, acc.at[pl.ds(off, _ZCH)])
            if with_counts:
                pltpu.sync_copy(zcnt_v, cacc.at[pl.ds(off, _ZCH)])
        if with_counts:
            pltpu.make_async_copy(ones_hbm, ones_v, zsem).wait()
        plsc.subcore_barrier()

        pltpu.make_async_copy(src_hbm.at[wid], src_v, isem).wait()
        pltpu.make_async_copy(dst_hbm.at[wid], dst_v, isem).wait()
        for b0 in range(la):
            pltpu.async_copy(tbl.at[src_v.at[b0]], rows_v.at[b0], gsems[b0])

        def step(t, _):
            for b in range(nbuf):
                j = nbuf * t + b
                b2 = (b + la) % nbuf
                pltpu.make_async_copy(
                    tbl.at[src_v.at[j]], rows_v.at[b], gsems[b]).wait()
                pltpu.async_copy(
                    rows_v.at[b], acc.at[dst_v.at[j]], ssems[b], add=True)
                if with_counts:
                    pltpu.sync_copy(ones_v, cacc.at[dst_v.at[j]], add=True)

                @pl.when((j >= nbuf - la) & (j + la < k))
                def _():
                    pltpu.make_async_copy(
                        rows_v.at[b2], acc.at[dst_v.at[0]],
                        ssems[b2]).wait()

                @pl.when(j + la < k)
                def _():
                    pltpu.async_copy(
                        tbl.at[src_v.at[j + la]], rows_v.at[b2], gsems[b2])
            return 0
        lax.fori_loop(0, k // nbuf, step, 0)
        for b in range(nbuf):
            pltpu.make_async_copy(
                rows_v.at[b], acc.at[dst_v.at[0]], ssems[b]).wait()
        plsc.subcore_barrier()

        sl = pl.ds(sid * rps, rps)
        for c in range(_NC):
            @pl.when(cid == c)
            def _():
                pltpu.sync_copy(acc.at[sl], out_hbm.at[sl, pl.ds(c * h, h)])
        if with_counts:
            pltpu.sync_copy(cacc.at[sl], cnt_hbm.at[cid, sl])

    return pl.kernel(body, out_type=out_type, mesh=mesh,
                     scratch_types=scratch,
                     compiler_params=pltpu.CompilerParams(
                         use_tc_tiling_on_sc=False))


_BN = 1000


def _dot_t(a, w):
    return lax.dot_general(a, w, (((1,), (1,)), ((), ())),
                           preferred_element_type=jnp.float32)


def _first_body(x_ref, wl_ref, wr_ref, bl_ref, p_ref, r_ref):
    x = x_ref[...]
    p_ref[...] = _dot_t(x, wl_ref[...])
    r_ref[...] = _dot_t(x, wr_ref[...]) + bl_ref[...]


def _mid_body(s_ref, ca_ref, cb_ref, r_ref, wl_ref, wr_ref, bl_ref,
              p_ref, rn_ref):
    cnt = ca_ref[0] + cb_ref[0]
    inv = 1.0 / jnp.maximum(cnt[:, 0:1], 1.0)
    sv = s_ref[...]
    hfeat = (sv[:, :64] + sv[:, 64:]) * inv + r_ref[...]
    hfeat = jnp.maximum(hfeat, 0.0)
    p_ref[...] = _dot_t(hfeat, wl_ref[...])
    rn_ref[...] = _dot_t(hfeat, wr_ref[...]) + bl_ref[...]


def _final_body(s_ref, ca_ref, cb_ref, r_ref, wlin_ref, blin_ref,
                z_ref):
    cnt = ca_ref[0] + cb_ref[0]
    inv = 1.0 / jnp.maximum(cnt[:, 0:1], 1.0)
    sv = s_ref[...]
    hfeat = (sv[:, :64] + sv[:, 64:]) * inv + r_ref[...]
    z_ref[...] = _dot_t(hfeat, wlin_ref[...]) + blin_ref[...]


def _row_spec(width):
    return pl.BlockSpec((_BN, width), lambda i: (i, 0))


def _part_spec(width, a):
    return pl.BlockSpec((1, _BN, width), lambda i, a=a: (a, i, 0))


def _full_spec(shape):
    return pl.BlockSpec(shape, lambda i: (0,) * len(shape))


@functools.cache
def _make_first_call(n, d, h):
    return pl.pallas_call(
        _first_body,
        grid=(n // _BN,),
        in_specs=[_row_spec(d), _full_spec((h, d)), _full_spec((h, d)),
                  _full_spec((1, h))],
        out_specs=[_row_spec(h), _row_spec(h)],
        out_shape=[jax.ShapeDtypeStruct((n, h), jnp.float32)] * 2,
    )


@functools.cache
def _make_mid_call(n, h):
    return pl.pallas_call(
        _mid_body,
        grid=(n // _BN,),
        in_specs=[_row_spec(2 * h), _part_spec(_L, 0),
                  _part_spec(_L, 1), _row_spec(h), _full_spec((h, h)),
                  _full_spec((h, h)), _full_spec((1, h))],
        out_specs=[_row_spec(h), _row_spec(h)],
        out_shape=[jax.ShapeDtypeStruct((n, h), jnp.float32)] * 2,
    )


@functools.cache
def _make_final_call(n, h, o):
    return pl.pallas_call(
        _final_body,
        grid=(n // _BN,),
        in_specs=[_row_spec(2 * h), _part_spec(_L, 0),
                  _part_spec(_L, 1), _row_spec(h), _full_spec((o, h)),
                  _full_spec((1, o))],
        out_specs=_row_spec(o),
        out_shape=jax.ShapeDtypeStruct((n, o), jnp.float32),
    )


@jax.jit
def kernel(x, edge_index, Wl1a, bl1a, Wr1a, Wl1b, bl1b, Wr1b,
           Wl2a, bl2a, Wr2a, Wl2b, bl2b, Wr2b, Wlin, blin):
    n, d = x.shape
    e = edge_index.shape[1]
    h = Wl1a.shape[0]
    o = Wlin.shape[0]

    ew = e // _NW
    src_r = edge_index[0].reshape(_NW, ew // _CHUNK, _CHUNK)
    dst_r = edge_index[1].reshape(_NW, ew // _CHUNK, _CHUNK)
    rps = _pad_n(n) // _NS
    zrow = jnp.zeros((rps, h), jnp.float32)
    z16 = jnp.zeros((rps, _L), jnp.float32)
    ones = jnp.ones((_CHUNK, _L), jnp.float32)

    seg1 = _make_seg_call(n, h, e, True)
    seg = _make_seg_call(n, h, e, False)
    first = _make_first_call(n, d, h)
    mid = _make_mid_call(n, h)
    final = _make_final_call(n, h, o)

    p1, r1 = first(x, Wl1a, Wr1a, bl1a[None])
    s1, c1 = seg1(p1, src_r, dst_r, zrow, z16, ones)

    p2, r2 = mid(s1, c1, c1, r1, Wl1b, Wr1b, bl1b[None])
    (s2,) = seg(p2, src_r, dst_r, zrow, z16, ones)
    p3, r3 = mid(s2, c1, c1, r2, Wl2a, Wr2a, bl2a[None])
    (s3,) = seg(p3, src_r, dst_r, zrow, z16, ones)
    p4, r4 = mid(s3, c1, c1, r3, Wl2b, Wr2b, bl2b[None])
    (s4,) = seg(p4, src_r, dst_r, zrow, z16, ones)
    z = final(s4, c1, c1, r4, Wlin, blin[None])
    return z
